# baseline (speedup 1.0000x reference)
import functools

import jax
import jax.numpy as jnp
from jax import lax
from jax.experimental import pallas as pl
from jax.experimental.pallas import tpu as pltpu
from jax.experimental.pallas import tpu_sc as plsc

NUM_NODES = 1000000
EMBED_DIM = 64
BATCH = 16384

_info = plsc.get_sparse_core_info()
_NC, _NS = _info.num_cores, _info.num_subcores
_NW = _NC * _NS
_B_PER_W = BATCH // _NW
_CHUNK = 32
_NCHUNK = _B_PER_W // _CHUNK
_NG = _CHUNK // 16

_mesh = plsc.VectorSubcoreMesh(core_axis_name="c", subcore_axis_name="s")


@functools.partial(
    pl.kernel,
    mesh=_mesh,
    out_type=jax.ShapeDtypeStruct((BATCH, EMBED_DIM), jnp.float32),
    scratch_types=[
        pltpu.VMEM((_B_PER_W,), jnp.int32),
        pltpu.VMEM((_CHUNK, 8, EMBED_DIM), jnp.float32),
        pltpu.VMEM((_B_PER_W, EMBED_DIM), jnp.float32),
        pltpu.SemaphoreType.DMA,
    ],
    compiler_params=pltpu.CompilerParams(needs_layout_passes=False),
)
def _gather_kernel(idx_hbm, table_hbm, out_hbm, idx_v, tiles_v, rows_v, sem):
    wid = lax.axis_index("s") * _NC + lax.axis_index("c")
    base = wid * _B_PER_W
    table_view = table_hbm.reshape(NUM_NODES // 8, 8, EMBED_DIM)
    pltpu.sync_copy(idx_hbm.at[pl.ds(base, _B_PER_W)], idx_v)
    iota = lax.iota(jnp.int32, 16)

    def chunk_body(j, carry):
        lo = j * _CHUNK
        for g in range(_NG):
            v = idx_v[pl.ds(lo + g * 16, 16)]
            t = lax.shift_right_logical(v, 3)
            for l in range(16):
                pltpu.make_async_copy(
                    table_view.at[pl.ds(t[l], 1)],
                    tiles_v.at[pl.ds(g * 16 + l, 1)],
                    sem,
                ).start()
        for g in range(_NG):
            for l in range(16):
                pltpu.make_async_copy(
                    table_view.at[pl.ds(0, 1)],
                    tiles_v.at[pl.ds(g * 16 + l, 1)],
                    sem,
                ).wait()
        for g in range(_NG):
            v = idx_v[pl.ds(lo + g * 16, 16)]
            s_vec = lax.bitwise_and(v, jnp.int32(7))
            i_vec = iota + g * 16
            out_row = iota + (lo + g * 16)
            for c in range(EMBED_DIM):
                c_vec = jnp.full((16,), c, jnp.int32)
                vals = plsc.load_gather(tiles_v, [i_vec, s_vec, c_vec])
                plsc.store_scatter(rows_v, [out_row, c_vec], vals)
        return carry

    lax.fori_loop(0, _NCHUNK, chunk_body, 0)
    pltpu.sync_copy(rows_v, out_hbm.at[pl.ds(base, _B_PER_W)])


def kernel(indices, weight):
    idx = indices.astype(jnp.int32)
    return _gather_kernel(idx, weight)
